# baseline (speedup 1.0000x reference)
import jax
import jax.numpy as jnp
from jax.experimental import pallas as pl


_BLOCK_S = 1024


def _add_body(x_ref, p_ref, o_ref):
    o_ref[...] = x_ref[...] + p_ref[...]


def kernel(inputs, pos_table):
    batch, seq_len, out_dim = inputs.shape
    ns = seq_len // _BLOCK_S
    flat = inputs.reshape(batch * seq_len, out_dim)
    out = pl.pallas_call(
        _add_body,
        grid=(ns, batch),
        in_specs=[
            pl.BlockSpec((_BLOCK_S, out_dim), lambda s, b, ns=ns: (b * ns + s, 0)),
            pl.BlockSpec((_BLOCK_S, out_dim), lambda s, b: (s, 0)),
        ],
        out_specs=pl.BlockSpec(
            (_BLOCK_S, out_dim), lambda s, b, ns=ns: (b * ns + s, 0)
        ),
        out_shape=jax.ShapeDtypeStruct(flat.shape, flat.dtype),
    )(flat, pos_table)
    return out.reshape(batch, seq_len, out_dim)


# speedup vs baseline: 1.0427x; 1.0427x over previous
import jax
import jax.numpy as jnp
from jax.experimental import pallas as pl


_BLOCK_S = 2048


def _add_body(x_ref, p_ref, o_ref):
    o_ref[...] = x_ref[...] + p_ref[...]


def kernel(inputs, pos_table):
    batch, seq_len, out_dim = inputs.shape
    ns = seq_len // _BLOCK_S
    flat = inputs.reshape(batch * seq_len, out_dim)
    out = pl.pallas_call(
        _add_body,
        grid=(ns, batch),
        in_specs=[
            pl.BlockSpec((_BLOCK_S, out_dim), lambda s, b, ns=ns: (b * ns + s, 0)),
            pl.BlockSpec((_BLOCK_S, out_dim), lambda s, b: (s, 0)),
        ],
        out_specs=pl.BlockSpec(
            (_BLOCK_S, out_dim), lambda s, b, ns=ns: (b * ns + s, 0)
        ),
        out_shape=jax.ShapeDtypeStruct(flat.shape, flat.dtype),
    )(flat, pos_table)
    return out.reshape(batch, seq_len, out_dim)
